# baseline (speedup 1.0000x reference)
import functools

import jax
import jax.numpy as jnp
from jax import lax
from jax.experimental import pallas as pl
from jax.experimental.pallas import tpu as pltpu
from jax.experimental.pallas import tpu_sc as plsc

N_NODES = 10000
N_EDGES = 160000
IN_DIM = 256
HALF = 128
MASK_RATE = 0.5

NS = 16
EDGES_PER_TILE = N_EDGES // NS
CHUNK = 80
NCHUNK = EDGES_PER_TILE // CHUNK
PH_CHUNKS = (32, 32, 32, 29)
PH_MAX = max(PH_CHUNKS)
STRIPE = 640
LAST_STRIPE = N_NODES - (NS - 1) * STRIPE

ROW_BLK = 1000
GRID = N_NODES // ROW_BLK


def _sc_segment_sum(xm2, src16, dst16, zeros_tile):
    mesh = plsc.VectorSubcoreMesh(core_axis_name="c", subcore_axis_name="s")

    @functools.partial(
        pl.kernel,
        out_type=jax.ShapeDtypeStruct((2 * N_NODES, HALF), jnp.float32),
        mesh=mesh,
        scratch_types=[
            pltpu.VMEM((PH_MAX, CHUNK), jnp.int32),
            pltpu.VMEM((PH_MAX, CHUNK), jnp.int32),
            pltpu.VMEM((CHUNK, HALF), jnp.float32),
            pltpu.VMEM((CHUNK, HALF), jnp.float32),
            pltpu.VMEM((CHUNK, HALF), jnp.float32),
            pltpu.VMEM((CHUNK, HALF), jnp.float32),
            pltpu.VMEM_SHARED((N_NODES, HALF), jnp.float32),
            pltpu.SemaphoreType.DMA,
            pltpu.SemaphoreType.DMA,
            pltpu.SemaphoreType.DMA,
            pltpu.SemaphoreType.DMA,
            pltpu.SemaphoreType.DMA,
            pltpu.SemaphoreType.DMA,
            pltpu.SemaphoreType.DMA,
            pltpu.SemaphoreType.DMA,
        ],
    )
    def k(xm_hbm, src_hbm, dst_hbm, zro_hbm, agg_hbm, src_v, dst_v,
          gb0, gb1, gb2, gb3, acc, sg0, sg1, sg2, sg3, ss0, ss1, ss2, ss3):
        c = lax.axis_index("c")
        s = lax.axis_index("s")

        @pl.when(s < NS - 1)
        def _():
            pltpu.sync_copy(zro_hbm, acc.at[pl.ds(s * STRIPE, STRIPE)])

        @pl.when(s == NS - 1)
        def _():
            pltpu.sync_copy(zro_hbm.at[pl.ds(0, LAST_STRIPE)],
                            acc.at[pl.ds((NS - 1) * STRIPE, LAST_STRIPE)])

        plsc.subcore_barrier()

        bufs = (gb0, gb1, gb2, gb3)
        sgs = (sg0, sg1, sg2, sg3)
        sss = (ss0, ss1, ss2, ss3)
        NB = 4

        def start_g(l, q):
            pltpu.async_copy(xm_hbm.at[src_v.at[l]], bufs[q], sgs[q])

        def wait_g(l, q):
            pltpu.make_async_copy(xm_hbm.at[src_v.at[l]], bufs[q], sgs[q]).wait()

        def start_s(l, q):
            pltpu.async_copy(bufs[q], acc.at[dst_v.at[l]], sss[q], add=True)

        def wait_s(l, q):
            pltpu.make_async_copy(bufs[q], acc.at[dst_v.at[l]], sss[q]).wait()

        def step(l, q, first=False, prefetch=True):
            wait_g(l, q)
            start_s(l, q)
            if prefetch:
                p = (q + NB - 1) % NB
                if not first:
                    wait_s(l - 1, p)
                start_g(l + NB - 1, p)

        def ring(m):
            for l in range(NB - 1):
                start_g(l, l)
            for l in range(NB):
                step(l, l, first=(l == 0))

            gmax = (m - 2 * NB + 1) // NB

            def group(g, carry):
                l0 = NB * g
                for q in range(NB):
                    step(l0 + q, q)
                return carry

            lax.fori_loop(1, gmax + 1, group, 0)
            for l in range(NB * (gmax + 1), m):
                step(l, l % NB, prefetch=(l <= m - NB))
            for l in range(m - NB, m):
                wait_s(l, l % NB)

        cbase = 0
        for ph, m in enumerate(PH_CHUNKS):
            pltpu.sync_copy(src_hbm.at[c, s, pl.ds(cbase, m)],
                            src_v.at[pl.ds(0, m)])
            pltpu.sync_copy(dst_hbm.at[s, pl.ds(cbase, m)],
                            dst_v.at[pl.ds(0, m)])
            ring(m)
            cbase += m

        plsc.subcore_barrier()

        @pl.when(s < NS - 1)
        def _():
            r0 = s * STRIPE
            pltpu.sync_copy(acc.at[pl.ds(r0, STRIPE)],
                            agg_hbm.at[pl.ds(c * N_NODES + r0, STRIPE)])

        @pl.when(s == NS - 1)
        def _():
            r0 = (NS - 1) * STRIPE
            pltpu.sync_copy(acc.at[pl.ds(r0, LAST_STRIPE)],
                            agg_hbm.at[pl.ds(c * N_NODES + r0, LAST_STRIPE)])

    return k(xm2, src16, dst16, zeros_tile)


def _mask_apply(x, mask_f, token):
    def body(x_ref, m_ref, t_ref, o_ref):
        o_ref[...] = jnp.where(m_ref[...] > 0.0, t_ref[...], x_ref[...])

    g = GRID
    return pl.pallas_call(
        body,
        grid=(2 * g,),
        in_specs=[
            pl.BlockSpec((ROW_BLK, HALF), lambda i: (i % g, i // g)),
            pl.BlockSpec((ROW_BLK, 1), lambda i: (i % g, 0)),
            pl.BlockSpec((1, HALF), lambda i: (0, i // g)),
        ],
        out_specs=pl.BlockSpec((ROW_BLK, HALF), lambda i: (i, 0)),
        out_shape=jax.ShapeDtypeStruct((2 * N_NODES, HALF), jnp.float32),
    )(x, mask_f, token)


def _dense_tail(xm2, agg2, x, mask_f, W_enc, b_enc, W1, b1, pa, W2, b2):
    def body(xl_ref, xr_ref, al_ref, ar_ref, x_ref, m_ref, we_ref, be_ref,
             w1_ref, b1_ref, pa_ref, w2_ref, b2_ref, ms_ref, nm_ref):
        xm = jnp.concatenate([xl_ref[...], xr_ref[...]], axis=1)
        ag = jnp.concatenate([al_ref[...], ar_ref[...]], axis=1)
        z = lax.dot(xm + ag, we_ref[...]) + be_ref[...]
        h = jnp.maximum(z, 0.0)
        t = lax.dot(h, w1_ref[...]) + b1_ref[...]
        a = pa_ref[0, 0]
        t = jnp.maximum(t, 0.0) + a * jnp.minimum(t, 0.0)
        xr = lax.dot(t, w2_ref[...]) + b2_ref[...]
        d = xr - x_ref[...]
        m = m_ref[...]
        part = jnp.sum(d * d * m)
        pm = jnp.sum(m)
        i = pl.program_id(0)

        @pl.when(i == 0)
        def _():
            ms_ref[0, 0] = part
            nm_ref[0, 0] = pm

        @pl.when(i > 0)
        def _():
            ms_ref[0, 0] += part
            nm_ref[0, 0] += pm

        @pl.when(i == GRID - 1)
        def _():
            ms_ref[0, 0] = ms_ref[0, 0] / (nm_ref[0, 0] * IN_DIM)

    full = lambda i: (0, 0)
    return pl.pallas_call(
        body,
        grid=(GRID,),
        in_specs=[
            pl.BlockSpec((ROW_BLK, HALF), lambda i: (i, 0)),
            pl.BlockSpec((ROW_BLK, HALF), lambda i: (GRID + i, 0)),
            pl.BlockSpec((ROW_BLK, HALF), lambda i: (i, 0)),
            pl.BlockSpec((ROW_BLK, HALF), lambda i: (GRID + i, 0)),
            pl.BlockSpec((ROW_BLK, IN_DIM), lambda i: (i, 0)),
            pl.BlockSpec((ROW_BLK, 1), lambda i: (i, 0)),
            pl.BlockSpec((IN_DIM, IN_DIM), full),
            pl.BlockSpec((1, IN_DIM), full),
            pl.BlockSpec((IN_DIM, IN_DIM), full),
            pl.BlockSpec((1, IN_DIM), full),
            pl.BlockSpec((1, 1), full),
            pl.BlockSpec((IN_DIM, IN_DIM), full),
            pl.BlockSpec((1, IN_DIM), full),
        ],
        out_specs=[pl.BlockSpec((1, 1), full, memory_space=pltpu.SMEM),
                   pl.BlockSpec((1, 1), full, memory_space=pltpu.SMEM)],
        out_shape=[jax.ShapeDtypeStruct((1, 1), jnp.float32),
                   jax.ShapeDtypeStruct((1, 1), jnp.float32)],
    )(xm2, xm2, agg2, agg2, x, mask_f, W_enc, b_enc, W1, b1, pa, W2, b2)


def kernel(x, edge_index, mask_token, W_enc, b_enc, W1, b1, prelu_a, W2, b2):
    N = x.shape[0]
    mask = jax.random.uniform(jax.random.key(42), (N,)) < MASK_RATE
    mask_f = mask.astype(jnp.float32)[:, None]

    xm2 = _mask_apply(x, mask_f, mask_token)

    e = edge_index.astype(jnp.int32)
    src16 = jnp.stack([e[0], e[0] + N_NODES]).reshape(2, NS, NCHUNK, CHUNK)
    dst16 = e[1].reshape(NS, NCHUNK, CHUNK)
    zeros_tile = jnp.zeros((STRIPE, HALF), jnp.float32)

    agg2 = _sc_segment_sum(xm2, src16, dst16, zeros_tile)

    ms, nm = _dense_tail(xm2, agg2, x, mask_f, W_enc,
                         b_enc.reshape(1, IN_DIM), W1, b1.reshape(1, IN_DIM),
                         prelu_a.reshape(1, 1), W2, b2.reshape(1, IN_DIM))
    return ms[0, 0]
